# Initial kernel scaffold; baseline (speedup 1.0000x reference)
import jax
import jax.numpy as jnp
from jax.experimental import pallas as pl


def kernel(x_ln1, v_first, Wq, Wk, Wv, E_v, W_att):
    raise NotImplementedError("write your pallas kernel here")


# speedup vs baseline: 19.5731x; 19.5731x over previous
import functools

import jax
import jax.numpy as jnp
from jax import lax
from jax.experimental import pallas as pl
from jax.experimental.pallas import tpu as pltpu
from jax.experimental.pallas import tpu_sc as plsc

_TT = 256
_CH = 128
_NC = 2
_NS = 16
_DP = 128


def _codes_body(x_ref, wq_ref, wk_ref, wv_ref, q_ref, k_ref, v_ref):
    TT, C = x_ref.shape
    M, D, K = wq_ref.shape
    lanes = lax.broadcasted_iota(jnp.int32, (TT, K), 1)
    for m in range(M):
        u = x_ref[:, m * D:(m + 1) * D]
        for w_ref, o_ref in ((wq_ref, q_ref), (wk_ref, k_ref), (wv_ref, v_ref)):
            logits = lax.dot_general(u, w_ref[m], (((1,), (0,)), ((), ())),
                                     preferred_element_type=jnp.float32)
            mx = jnp.max(logits, axis=1, keepdims=True)
            idx = jnp.min(jnp.where(logits == mx, lanes, K), axis=1)
            o_ref[m, :] = idx.astype(jnp.int32)


def _compute_codes(x, Wq, Wk, Wv):
    T, C = x.shape
    M, D, K = Wq.shape
    grid = (T // _TT,)
    wspec = pl.BlockSpec((M, D, K), lambda i: (0, 0, 0))
    ospec = pl.BlockSpec((16, _TT), lambda i: (0, i))
    oshape = jax.ShapeDtypeStruct((16, T), jnp.int32)
    return pl.pallas_call(
        _codes_body,
        grid=grid,
        in_specs=[pl.BlockSpec((_TT, C), lambda i: (i, 0)), wspec, wspec, wspec],
        out_specs=[ospec, ospec, ospec],
        out_shape=[oshape, oshape, oshape],
    )(x, Wq, Wk, Wv)


def _out_body(x_ref, w_ref, v_ref, o_ref):
    M, TT, DP = v_ref.shape
    C = x_ref.shape[1]
    D = C // M
    y = lax.dot_general(x_ref[...], w_ref[...], (((1,), (0,)), ((), ())),
                        preferred_element_type=jnp.float32)
    for m in range(M):
        o_ref[:, m * D:(m + 1) * D] = y[:, m * D:(m + 1) * D] + v_ref[m][:, :D]


def _compute_out(x, W_att, vout):
    T, C = x.shape
    M, _, DP = vout.shape
    return pl.pallas_call(
        _out_body,
        grid=(T // _TT,),
        in_specs=[pl.BlockSpec((_TT, C), lambda i: (i, 0)),
                  pl.BlockSpec((C, C), lambda i: (0, 0)),
                  pl.BlockSpec((M, _TT, DP), lambda i: (0, i, 0))],
        out_specs=pl.BlockSpec((_TT, C), lambda i: (i, 0)),
        out_shape=jax.ShapeDtypeStruct((T, C), jnp.float32),
    )(x, W_att, vout)


def _sc_vout(qk, kk, vk, ef, T, C, M, D, K, KV):
    mesh = plsc.VectorSubcoreMesh(core_axis_name="c", subcore_axis_name="s")

    @functools.partial(
        pl.kernel,
        mesh=mesh,
        compiler_params=pltpu.CompilerParams(needs_layout_passes=False),
        out_type=jax.ShapeDtypeStruct((M, T, _DP), jnp.float32),
        scratch_types=[
            pltpu.VMEM((T,), jnp.int32),
            pltpu.VMEM((T,), jnp.int32),
            pltpu.VMEM((T,), jnp.int32),
            pltpu.VMEM((K,), jnp.int32),
            pltpu.VMEM((T,), jnp.int32),
            pltpu.VMEM((_CH, _DP), jnp.float32),
            pltpu.SemaphoreType.DMA,
        ],
    )
    def sc_kernel(qk_hbm, kk_hbm, vk_hbm, ef_hbm, out_hbm,
                  qv, kv, vv, tbl, idxv, rows, sem):
        w = lax.axis_index("s") * _NC + lax.axis_index("c")

        @pl.when(w < M)
        def _():
            m = w
            pltpu.sync_copy(qk_hbm.at[m], qv)
            pltpu.sync_copy(kk_hbm.at[m], kv)
            pltpu.sync_copy(vk_hbm.at[m], vv)

            neg = jnp.full((16,), -1, jnp.int32)

            def init_step(i, carry):
                tbl[pl.ds(i * 16, 16)] = neg
                return carry

            lax.fori_loop(0, K // 16, init_step, 0)

            base = m * (KV + 1)
            iota = lax.iota(jnp.int32, 16)

            def scan_step(ci, carry):
                t0 = ci * 16
                qv16 = qv[pl.ds(t0, 16)]
                kv16 = kv[pl.ds(t0, 16)]
                tau = plsc.load_gather(tbl, [qv16])
                keep = jnp.ones((16,), jnp.bool_)
                for j in range(16):
                    kjb = jnp.full((16,), kv16[j])
                    tau = jnp.where((qv16 == kjb) & (iota >= j), t0 + j, tau)
                    dup = jnp.any((kv16 == kjb) & (iota > j))
                    keep = jnp.where(iota == j, jnp.logical_not(dup), keep)
                plsc.store_scatter(tbl, [kv16], t0 + iota, mask=keep)
                valid = tau >= 0
                vi = plsc.load_gather(vv, [jnp.maximum(tau, 0)])
                idxv[pl.ds(t0, 16)] = jnp.where(valid, base + vi + 1, base)
                return carry

            lax.fori_loop(0, T // 16, scan_step, 0)

            def chunk_step(ci, carry):
                t0 = ci * _CH
                pltpu.async_copy(ef_hbm.at[idxv.at[pl.ds(t0, _CH)]], rows, sem).wait()
                pltpu.sync_copy(rows, out_hbm.at[m, pl.ds(t0, _CH)])
                return carry

            lax.fori_loop(0, T // _CH, chunk_step, 0)

    return sc_kernel(qk, kk, vk, ef)


def kernel(x_ln1, v_first, Wq, Wk, Wv, E_v, W_att):
    x = x_ln1[0]
    T, C = x.shape
    M, D, K = Wq.shape
    KV = Wv.shape[2]
    qk, kk, vk = _compute_codes(x, Wq, Wk, Wv)
    ef = jnp.pad(E_v, ((0, 0), (0, 0), (0, _DP - D))).reshape(M * (KV + 1), _DP)
    vout = _sc_vout(qk, kk, vk, ef, T, C, M, D, K, KV)
    y = _compute_out(x, W_att, vout)
    return (y[None], v_first)


# speedup vs baseline: 23.0948x; 1.1799x over previous
import functools

import jax
import jax.numpy as jnp
from jax import lax
from jax.experimental import pallas as pl
from jax.experimental.pallas import tpu as pltpu
from jax.experimental.pallas import tpu_sc as plsc

_TT = 256
_CH = 128
_NC = 2
_NS = 16
_DP = 128


def _codes_body(x_ref, wq_ref, wk_ref, wv_ref, q_ref, k_ref, v_ref):
    TT, C = x_ref.shape
    M, K, D = wq_ref.shape
    rows_f = lax.broadcasted_iota(jnp.int32, (K, TT), 0).astype(jnp.float32)
    for m in range(M):
        u = x_ref[:, m * D:(m + 1) * D]
        for w_ref, o_ref in ((wq_ref, q_ref), (wk_ref, k_ref), (wv_ref, v_ref)):
            logits = lax.dot_general(w_ref[m], u, (((1,), (1,)), ((), ())),
                                     preferred_element_type=jnp.float32)
            v, ix = logits, rows_f
            n = K
            while n > 1:
                h = n // 2
                keep = v[:h] >= v[h:n]
                v = jnp.where(keep, v[:h], v[h:n])
                ix = jnp.where(keep, ix[:h], ix[h:n])
                n = h
            o_ref[m, :] = ix[0].astype(jnp.int32)


def _compute_codes(x, Wq, Wk, Wv):
    T, C = x.shape
    M, D, K = Wq.shape
    grid = (T // _TT,)
    wspec = pl.BlockSpec((M, K, D), lambda i: (0, 0, 0))
    ospec = pl.BlockSpec((16, _TT), lambda i: (0, i))
    oshape = jax.ShapeDtypeStruct((16, T), jnp.int32)
    wqt = Wq.transpose(0, 2, 1)
    wkt = Wk.transpose(0, 2, 1)
    wvt = Wv.transpose(0, 2, 1)
    return pl.pallas_call(
        _codes_body,
        grid=grid,
        in_specs=[pl.BlockSpec((_TT, C), lambda i: (i, 0)), wspec, wspec, wspec],
        out_specs=[ospec, ospec, ospec],
        out_shape=[oshape, oshape, oshape],
    )(x, wqt, wkt, wvt)


def _out_body(x_ref, w_ref, v_ref, o_ref):
    M, TT, DP = v_ref.shape
    C = x_ref.shape[1]
    D = C // M
    y = lax.dot_general(x_ref[...], w_ref[...], (((1,), (0,)), ((), ())),
                        preferred_element_type=jnp.float32)
    for m in range(M):
        o_ref[:, m * D:(m + 1) * D] = y[:, m * D:(m + 1) * D] + v_ref[m][:, :D]


def _compute_out(x, W_att, vout):
    T, C = x.shape
    M, _, DP = vout.shape
    return pl.pallas_call(
        _out_body,
        grid=(T // _TT,),
        in_specs=[pl.BlockSpec((_TT, C), lambda i: (i, 0)),
                  pl.BlockSpec((C, C), lambda i: (0, 0)),
                  pl.BlockSpec((M, _TT, DP), lambda i: (0, i, 0))],
        out_specs=pl.BlockSpec((_TT, C), lambda i: (i, 0)),
        out_shape=jax.ShapeDtypeStruct((T, C), jnp.float32),
    )(x, W_att, vout)


def _sc_vout(qk, kk, vk, ef, T, C, M, D, K, KV):
    mesh = plsc.VectorSubcoreMesh(core_axis_name="c", subcore_axis_name="s")

    @functools.partial(
        pl.kernel,
        mesh=mesh,
        compiler_params=pltpu.CompilerParams(needs_layout_passes=False),
        out_type=jax.ShapeDtypeStruct((M, T, _DP), jnp.float32),
        scratch_types=[
            pltpu.VMEM((T,), jnp.int32),
            pltpu.VMEM((T,), jnp.int32),
            pltpu.VMEM((T,), jnp.int32),
            pltpu.VMEM((K,), jnp.int32),
            pltpu.VMEM((T,), jnp.int32),
            pltpu.VMEM((_CH, _DP), jnp.float32),
            pltpu.VMEM((_CH, _DP), jnp.float32),
            pltpu.SemaphoreType.DMA,
            pltpu.SemaphoreType.DMA,
            pltpu.SemaphoreType.DMA,
            pltpu.SemaphoreType.DMA,
        ],
    )
    def sc_kernel(qk_hbm, kk_hbm, vk_hbm, ef_hbm, out_hbm,
                  qv, kv, vv, tbl, idxv, rows0, rows1,
                  gsem0, gsem1, wsem0, wsem1):
        w = lax.axis_index("s") * _NC + lax.axis_index("c")

        @pl.when(w < M)
        def _():
            m = w
            pltpu.sync_copy(qk_hbm.at[m], qv)
            pltpu.sync_copy(kk_hbm.at[m], kv)
            pltpu.sync_copy(vk_hbm.at[m], vv)

            neg = jnp.full((16,), -1, jnp.int32)

            def init_step(i, carry):
                tbl[pl.ds(i * 16, 16)] = neg
                return carry

            lax.fori_loop(0, K // 16, init_step, 0)

            base = m * (KV + 1)
            iota = lax.iota(jnp.int32, 16)

            def scan_step(ci, carry):
                t0 = ci * 16
                qv16 = qv[pl.ds(t0, 16)]
                kv16 = kv[pl.ds(t0, 16)]
                tau = plsc.load_gather(tbl, [qv16])
                keep = jnp.ones((16,), jnp.bool_)
                for j in range(16):
                    kjb = jnp.full((16,), kv16[j])
                    tau = jnp.where((qv16 == kjb) & (iota >= j), t0 + j, tau)
                    dup = jnp.any((kv16 == kjb) & (iota > j))
                    keep = jnp.where(iota == j, jnp.logical_not(dup), keep)
                plsc.store_scatter(tbl, [kv16], t0 + iota, mask=keep)
                valid = tau >= 0
                vi = plsc.load_gather(vv, [jnp.maximum(tau, 0)])
                idxv[pl.ds(t0, 16)] = jnp.where(valid, base + vi + 1, base)
                return carry

            lax.fori_loop(0, T // 16, scan_step, 0)

            nch = T // _CH
            bufs = (rows0, rows1)
            gsems = (gsem0, gsem1)
            wsems = (wsem0, wsem1)

            def fire_gather(ci):
                return pltpu.async_copy(
                    ef_hbm.at[idxv.at[pl.ds(ci * _CH, _CH)]],
                    bufs[ci % 2], gsems[ci % 2])

            def fire_write(ci):
                return pltpu.async_copy(
                    bufs[ci % 2], out_hbm.at[m, pl.ds(ci * _CH, _CH)],
                    wsems[ci % 2])

            gcop = [None] * nch
            wcop = [None] * nch
            gcop[0] = fire_gather(0)
            for ci in range(nch):
                gcop[ci].wait()
                wcop[ci] = fire_write(ci)
                if ci + 1 < nch:
                    if ci >= 1:
                        wcop[ci - 1].wait()
                    gcop[ci + 1] = fire_gather(ci + 1)
            wcop[nch - 1].wait()

    return sc_kernel(qk, kk, vk, ef)


def kernel(x_ln1, v_first, Wq, Wk, Wv, E_v, W_att):
    x = x_ln1[0]
    T, C = x.shape
    M, D, K = Wq.shape
    KV = Wv.shape[2]
    qk, kk, vk = _compute_codes(x, Wq, Wk, Wv)
    ef = jnp.pad(E_v, ((0, 0), (0, 0), (0, _DP - D))).reshape(M * (KV + 1), _DP)
    vout = _sc_vout(qk, kk, vk, ef, T, C, M, D, K, KV)
    y = _compute_out(x, W_att, vout)
    return (y[None], v_first)


# speedup vs baseline: 24.0425x; 1.0410x over previous
import functools

import jax
import jax.numpy as jnp
from jax import lax
from jax.experimental import pallas as pl
from jax.experimental.pallas import tpu as pltpu
from jax.experimental.pallas import tpu_sc as plsc

_TT = 256
_CH = 128
_NC = 2
_NS = 16
_DP = 128


def _codes_body(x_ref, wq_ref, wk_ref, wv_ref, q_ref, k_ref, v_ref):
    TT, C = x_ref.shape
    M, K, D = wq_ref.shape
    rows_f = lax.broadcasted_iota(jnp.int32, (K, TT), 0).astype(jnp.float32)
    for m in range(M):
        u = x_ref[:, m * D:(m + 1) * D]
        for w_ref, o_ref in ((wq_ref, q_ref), (wk_ref, k_ref), (wv_ref, v_ref)):
            logits = lax.dot_general(w_ref[m], u, (((1,), (1,)), ((), ())),
                                     preferred_element_type=jnp.float32)
            v, ix = logits, rows_f
            n = K
            while n > 1:
                h = n // 2
                keep = v[:h] >= v[h:n]
                v = jnp.where(keep, v[:h], v[h:n])
                ix = jnp.where(keep, ix[:h], ix[h:n])
                n = h
            o_ref[m, :] = ix[0].astype(jnp.int32)


def _compute_codes(x, Wq, Wk, Wv):
    T, C = x.shape
    M, D, K = Wq.shape
    grid = (T // _TT,)
    wspec = pl.BlockSpec((M, K, D), lambda i: (0, 0, 0))
    ospec = pl.BlockSpec((16, _TT), lambda i: (0, i))
    oshape = jax.ShapeDtypeStruct((16, T), jnp.int32)
    wqt = Wq.transpose(0, 2, 1)
    wkt = Wk.transpose(0, 2, 1)
    wvt = Wv.transpose(0, 2, 1)
    return pl.pallas_call(
        _codes_body,
        grid=grid,
        in_specs=[pl.BlockSpec((_TT, C), lambda i: (i, 0)), wspec, wspec, wspec],
        out_specs=[ospec, ospec, ospec],
        out_shape=[oshape, oshape, oshape],
    )(x, wqt, wkt, wvt)


def _out_body(x_ref, w_ref, v_ref, o_ref):
    M, TT, DP = v_ref.shape
    C = x_ref.shape[1]
    D = C // M
    y = lax.dot_general(x_ref[...], w_ref[...], (((1,), (0,)), ((), ())),
                        preferred_element_type=jnp.float32)
    for m in range(M):
        o_ref[:, m * D:(m + 1) * D] = y[:, m * D:(m + 1) * D] + v_ref[m][:, :D]


def _compute_out(x, W_att, vout):
    T, C = x.shape
    M, _, DP = vout.shape
    return pl.pallas_call(
        _out_body,
        grid=(T // _TT,),
        in_specs=[pl.BlockSpec((_TT, C), lambda i: (i, 0)),
                  pl.BlockSpec((C, C), lambda i: (0, 0)),
                  pl.BlockSpec((M, _TT, DP), lambda i: (0, i, 0))],
        out_specs=pl.BlockSpec((_TT, C), lambda i: (i, 0)),
        out_shape=jax.ShapeDtypeStruct((T, C), jnp.float32),
    )(x, W_att, vout)


def _sc_vout(qk, kk, vk, ef, T, C, M, D, K, KV):
    mesh = plsc.VectorSubcoreMesh(core_axis_name="c", subcore_axis_name="s")

    @functools.partial(
        pl.kernel,
        mesh=mesh,
        compiler_params=pltpu.CompilerParams(needs_layout_passes=False),
        out_type=jax.ShapeDtypeStruct((M, T, _DP), jnp.float32),
        scratch_types=[
            pltpu.VMEM((T,), jnp.int32),
            pltpu.VMEM((T,), jnp.int32),
            pltpu.VMEM((T,), jnp.int32),
            pltpu.VMEM((K,), jnp.int32),
            pltpu.VMEM((T,), jnp.int32),
            pltpu.VMEM((_CH, _DP), jnp.float32),
            pltpu.VMEM((_CH, _DP), jnp.float32),
            pltpu.SemaphoreType.DMA,
            pltpu.SemaphoreType.DMA,
            pltpu.SemaphoreType.DMA,
            pltpu.SemaphoreType.DMA,
        ],
    )
    def sc_kernel(qk_hbm, kk_hbm, vk_hbm, ef_hbm, out_hbm,
                  qv, kv, vv, tbl, idxv, rows0, rows1,
                  gsem0, gsem1, wsem0, wsem1):
        w = lax.axis_index("s") * _NC + lax.axis_index("c")

        @pl.when(w < M)
        def _():
            m = w
            pltpu.sync_copy(qk_hbm.at[m], qv)
            pltpu.sync_copy(kk_hbm.at[m], kv)
            pltpu.sync_copy(vk_hbm.at[m], vv)

            neg = jnp.full((16,), -1, jnp.int32)

            def init_step(i, carry):
                tbl[pl.ds(i * 16, 16)] = neg
                return carry

            lax.fori_loop(0, K // 16, init_step, 0)

            base = m * (KV + 1)
            iota = lax.iota(jnp.int32, 16)

            def scan_step(ci, carry):
                t0 = ci * 16
                qv16 = qv[pl.ds(t0, 16)]
                kv16 = kv[pl.ds(t0, 16)]
                tau = plsc.load_gather(tbl, [qv16])
                keep = jnp.ones((16,), jnp.bool_)
                for j in range(16):
                    kjb = jnp.full((16,), kv16[j])
                    tau = jnp.where((qv16 == kjb) & (iota >= j), t0 + j, tau)
                    ndup = plsc.all_reduce_population_count(
                        (kv16 == kjb) & (iota > j))
                    keep = jnp.where(iota == j, ndup == 0, keep)
                plsc.store_scatter(tbl, [kv16], t0 + iota, mask=keep)
                valid = tau >= 0
                vi = plsc.load_gather(vv, [jnp.maximum(tau, 0)])
                idxv[pl.ds(t0, 16)] = jnp.where(valid, base + vi + 1, base)
                return carry

            lax.fori_loop(0, T // 16, scan_step, 0)

            nch = T // _CH
            bufs = (rows0, rows1)
            gsems = (gsem0, gsem1)
            wsems = (wsem0, wsem1)

            def fire_gather(ci):
                return pltpu.async_copy(
                    ef_hbm.at[idxv.at[pl.ds(ci * _CH, _CH)]],
                    bufs[ci % 2], gsems[ci % 2])

            def fire_write(ci):
                return pltpu.async_copy(
                    bufs[ci % 2], out_hbm.at[m, pl.ds(ci * _CH, _CH)],
                    wsems[ci % 2])

            gcop = [None] * nch
            wcop = [None] * nch
            gcop[0] = fire_gather(0)
            for ci in range(nch):
                gcop[ci].wait()
                wcop[ci] = fire_write(ci)
                if ci + 1 < nch:
                    if ci >= 1:
                        wcop[ci - 1].wait()
                    gcop[ci + 1] = fire_gather(ci + 1)
            wcop[nch - 1].wait()

    return sc_kernel(qk, kk, vk, ef)


def kernel(x_ln1, v_first, Wq, Wk, Wv, E_v, W_att):
    x = x_ln1[0]
    T, C = x.shape
    M, D, K = Wq.shape
    KV = Wv.shape[2]
    qk, kk, vk = _compute_codes(x, Wq, Wk, Wv)
    ef = jnp.pad(E_v, ((0, 0), (0, 0), (0, _DP - D))).reshape(M * (KV + 1), _DP)
    vout = _sc_vout(qk, kk, vk, ef, T, C, M, D, K, KV)
    y = _compute_out(x, W_att, vout)
    return (y[None], v_first)


# speedup vs baseline: 24.3803x; 1.0140x over previous
import functools

import jax
import jax.numpy as jnp
from jax import lax
from jax.experimental import pallas as pl
from jax.experimental.pallas import tpu as pltpu
from jax.experimental.pallas import tpu_sc as plsc

_TT = 256
_CH = 128
_NC = 2
_NS = 16
_DP = 128


def _codes_body(x_ref, wq_ref, wk_ref, wv_ref, q_ref, k_ref, v_ref):
    TT, C = x_ref.shape
    M, K, D = wq_ref.shape
    CK = 128
    rows_f = lax.broadcasted_iota(jnp.int32, (CK, TT), 0).astype(jnp.float32)

    def fold(v, ix, n, stop):
        while n > stop:
            h = n // 2
            keep = v[:h] >= v[h:n]
            v = jnp.where(keep, v[:h], v[h:n])
            ix = jnp.where(keep, ix[:h], ix[h:n])
            n = h
        return v, ix

    for m in range(M):
        u = x_ref[:, m * D:(m + 1) * D]
        for w_ref, o_ref in ((wq_ref, q_ref), (wk_ref, k_ref), (wv_ref, v_ref)):
            pv, pi = None, None
            for c in range(K // CK):
                logits = lax.dot_general(w_ref[m, c * CK:(c + 1) * CK], u,
                                         (((1,), (1,)), ((), ())),
                                         preferred_element_type=jnp.float32)
                v, ix = fold(logits, rows_f + float(c * CK), CK, 8)
                if pv is None:
                    pv, pi = v, ix
                else:
                    keep = pv >= v
                    pv = jnp.where(keep, pv, v)
                    pi = jnp.where(keep, pi, ix)
            v, ix = fold(pv, pi, 8, 1)
            o_ref[m, :] = ix[0].astype(jnp.int32)


def _compute_codes(x, Wq, Wk, Wv):
    T, C = x.shape
    M, D, K = Wq.shape
    grid = (T // _TT,)
    wspec = pl.BlockSpec((M, K, D), lambda i: (0, 0, 0))
    ospec = pl.BlockSpec((16, _TT), lambda i: (0, i))
    oshape = jax.ShapeDtypeStruct((16, T), jnp.int32)
    wqt = Wq.transpose(0, 2, 1)
    wkt = Wk.transpose(0, 2, 1)
    wvt = Wv.transpose(0, 2, 1)
    return pl.pallas_call(
        _codes_body,
        grid=grid,
        in_specs=[pl.BlockSpec((_TT, C), lambda i: (i, 0)), wspec, wspec, wspec],
        out_specs=[ospec, ospec, ospec],
        out_shape=[oshape, oshape, oshape],
    )(x, wqt, wkt, wvt)


def _out_body(x_ref, w_ref, v_ref, o_ref):
    M, TT, DP = v_ref.shape
    C = x_ref.shape[1]
    D = C // M
    y = lax.dot_general(x_ref[...], w_ref[...], (((1,), (0,)), ((), ())),
                        preferred_element_type=jnp.float32)
    for m in range(M):
        o_ref[:, m * D:(m + 1) * D] = y[:, m * D:(m + 1) * D] + v_ref[m][:, :D]


def _compute_out(x, W_att, vout):
    T, C = x.shape
    M, _, DP = vout.shape
    return pl.pallas_call(
        _out_body,
        grid=(T // _TT,),
        in_specs=[pl.BlockSpec((_TT, C), lambda i: (i, 0)),
                  pl.BlockSpec((C, C), lambda i: (0, 0)),
                  pl.BlockSpec((M, _TT, DP), lambda i: (0, i, 0))],
        out_specs=pl.BlockSpec((_TT, C), lambda i: (i, 0)),
        out_shape=jax.ShapeDtypeStruct((T, C), jnp.float32),
    )(x, W_att, vout)


def _sc_vout(qk, kk, vk, ef, T, C, M, D, K, KV):
    mesh = plsc.VectorSubcoreMesh(core_axis_name="c", subcore_axis_name="s")

    @functools.partial(
        pl.kernel,
        mesh=mesh,
        compiler_params=pltpu.CompilerParams(needs_layout_passes=False),
        out_type=jax.ShapeDtypeStruct((M, T, _DP), jnp.float32),
        scratch_types=[
            pltpu.VMEM((T,), jnp.int32),
            pltpu.VMEM((T,), jnp.int32),
            pltpu.VMEM((T,), jnp.int32),
            pltpu.VMEM((K,), jnp.int32),
            pltpu.VMEM((T,), jnp.int32),
            pltpu.VMEM((_CH, _DP), jnp.float32),
            pltpu.VMEM((_CH, _DP), jnp.float32),
            pltpu.SemaphoreType.DMA,
            pltpu.SemaphoreType.DMA,
            pltpu.SemaphoreType.DMA,
            pltpu.SemaphoreType.DMA,
        ],
    )
    def sc_kernel(qk_hbm, kk_hbm, vk_hbm, ef_hbm, out_hbm,
                  qv, kv, vv, tbl, idxv, rows0, rows1,
                  gsem0, gsem1, wsem0, wsem1):
        w = lax.axis_index("s") * _NC + lax.axis_index("c")

        @pl.when(w < M)
        def _():
            m = w
            pltpu.sync_copy(qk_hbm.at[m], qv)
            pltpu.sync_copy(kk_hbm.at[m], kv)
            pltpu.sync_copy(vk_hbm.at[m], vv)

            neg = jnp.full((16,), -1, jnp.int32)

            def init_step(i, carry):
                tbl[pl.ds(i * 16, 16)] = neg
                return carry

            lax.fori_loop(0, K // 16, init_step, 0)

            base = m * (KV + 1)
            iota = lax.iota(jnp.int32, 16)

            def scan_step(ci, carry):
                t0 = ci * 16
                qv16 = qv[pl.ds(t0, 16)]
                kv16 = kv[pl.ds(t0, 16)]
                tau = plsc.load_gather(tbl, [qv16])
                keep = jnp.ones((16,), jnp.bool_)
                for j in range(16):
                    kjb = jnp.full((16,), kv16[j])
                    tau = jnp.where((qv16 == kjb) & (iota >= j), t0 + j, tau)
                    ndup = plsc.all_reduce_population_count(
                        (kv16 == kjb) & (iota > j))
                    keep = jnp.where(iota == j, ndup == 0, keep)
                plsc.store_scatter(tbl, [kv16], t0 + iota, mask=keep)
                valid = tau >= 0
                vi = plsc.load_gather(vv, [jnp.maximum(tau, 0)])
                idxv[pl.ds(t0, 16)] = jnp.where(valid, base + vi + 1, base)
                return carry

            lax.fori_loop(0, T // 16, scan_step, 0)

            nch = T // _CH
            bufs = (rows0, rows1)
            gsems = (gsem0, gsem1)
            wsems = (wsem0, wsem1)

            def fire_gather(ci):
                return pltpu.async_copy(
                    ef_hbm.at[idxv.at[pl.ds(ci * _CH, _CH)]],
                    bufs[ci % 2], gsems[ci % 2])

            def fire_write(ci):
                return pltpu.async_copy(
                    bufs[ci % 2], out_hbm.at[m, pl.ds(ci * _CH, _CH)],
                    wsems[ci % 2])

            gcop = [None] * nch
            wcop = [None] * nch
            gcop[0] = fire_gather(0)
            for ci in range(nch):
                gcop[ci].wait()
                wcop[ci] = fire_write(ci)
                if ci + 1 < nch:
                    if ci >= 1:
                        wcop[ci - 1].wait()
                    gcop[ci + 1] = fire_gather(ci + 1)
            wcop[nch - 1].wait()

    return sc_kernel(qk, kk, vk, ef)


def kernel(x_ln1, v_first, Wq, Wk, Wv, E_v, W_att):
    x = x_ln1[0]
    T, C = x.shape
    M, D, K = Wq.shape
    KV = Wv.shape[2]
    qk, kk, vk = _compute_codes(x, Wq, Wk, Wv)
    ef = jnp.pad(E_v, ((0, 0), (0, 0), (0, _DP - D))).reshape(M * (KV + 1), _DP)
    vout = _sc_vout(qk, kk, vk, ef, T, C, M, D, K, KV)
    y = _compute_out(x, W_att, vout)
    return (y[None], v_first)


# speedup vs baseline: 26.7586x; 1.0976x over previous
import functools

import jax
import jax.numpy as jnp
from jax import lax
from jax.experimental import pallas as pl
from jax.experimental.pallas import tpu as pltpu
from jax.experimental.pallas import tpu_sc as plsc

_TT = 256
_CH = 128
_NC = 2
_NS = 16
_DP = 128


def _codes_body(x_ref, wq_ref, wk_ref, wv_ref, q_ref, k_ref, v_ref):
    TT, C = x_ref.shape
    M, D, K = wq_ref.shape
    CK = 128
    rows_f = lax.broadcasted_iota(jnp.int32, (CK, TT), 0).astype(jnp.float32)

    def fold(v, ix, n, stop):
        while n > stop:
            h = n // 2
            keep = v[:h] >= v[h:n]
            v = jnp.where(keep, v[:h], v[h:n])
            ix = jnp.where(keep, ix[:h], ix[h:n])
            n = h
        return v, ix

    for m in range(M):
        u = x_ref[:, m * D:(m + 1) * D]
        for w_ref, o_ref in ((wq_ref, q_ref), (wk_ref, k_ref), (wv_ref, v_ref)):
            pv, pi = None, None
            for c in range(K // CK):
                logits = lax.dot_general(w_ref[m, :, c * CK:(c + 1) * CK], u,
                                         (((0,), (1,)), ((), ())),
                                         preferred_element_type=jnp.float32)
                v, ix = fold(logits, rows_f + float(c * CK), CK, 8)
                if pv is None:
                    pv, pi = v, ix
                else:
                    keep = pv >= v
                    pv = jnp.where(keep, pv, v)
                    pi = jnp.where(keep, pi, ix)
            v, ix = fold(pv, pi, 8, 1)
            o_ref[m, :] = ix[0].astype(jnp.int32)


def _compute_codes(x, Wq, Wk, Wv):
    T, C = x.shape
    M, D, K = Wq.shape
    grid = (T // _TT,)
    wspec = pl.BlockSpec((M, D, K), lambda i: (0, 0, 0))
    ospec = pl.BlockSpec((16, _TT), lambda i: (0, i))
    oshape = jax.ShapeDtypeStruct((16, T), jnp.int32)
    return pl.pallas_call(
        _codes_body,
        grid=grid,
        in_specs=[pl.BlockSpec((_TT, C), lambda i: (i, 0)), wspec, wspec, wspec],
        out_specs=[ospec, ospec, ospec],
        out_shape=[oshape, oshape, oshape],
    )(x, Wq, Wk, Wv)


def _out_body(x_ref, w_ref, v_ref, o_ref):
    M, TT, DP = v_ref.shape
    C = x_ref.shape[1]
    D = C // M
    y = lax.dot_general(x_ref[...], w_ref[...], (((1,), (0,)), ((), ())),
                        preferred_element_type=jnp.float32)
    for m in range(M):
        o_ref[:, m * D:(m + 1) * D] = y[:, m * D:(m + 1) * D] + v_ref[m][:, :D]


def _compute_out(x, W_att, vout):
    T, C = x.shape
    M, _, DP = vout.shape
    return pl.pallas_call(
        _out_body,
        grid=(T // _TT,),
        in_specs=[pl.BlockSpec((_TT, C), lambda i: (i, 0)),
                  pl.BlockSpec((C, C), lambda i: (0, 0)),
                  pl.BlockSpec((M, _TT, DP), lambda i: (0, i, 0))],
        out_specs=pl.BlockSpec((_TT, C), lambda i: (i, 0)),
        out_shape=jax.ShapeDtypeStruct((T, C), jnp.float32),
    )(x, W_att, vout)


def _ef_body(e_ref, o_ref):
    _, R, D = e_ref.shape
    RP, DP = o_ref.shape
    a = jnp.concatenate([e_ref[0], jnp.zeros((RP - R, D), jnp.float32)], 0)
    o_ref[...] = jnp.concatenate([a, jnp.zeros((RP, DP - D), jnp.float32)], 1)


def _prep_ef(E_v, RP):
    M, R, D = E_v.shape
    return pl.pallas_call(
        _ef_body,
        grid=(M,),
        in_specs=[pl.BlockSpec((1, R, D), lambda i: (i, 0, 0))],
        out_specs=pl.BlockSpec((RP, _DP), lambda i: (i, 0)),
        out_shape=jax.ShapeDtypeStruct((M * RP, _DP), jnp.float32),
    )(E_v)


def _sc_vout(qk, kk, vk, ef, T, C, M, D, K, RP):
    mesh = plsc.VectorSubcoreMesh(core_axis_name="c", subcore_axis_name="s")

    @functools.partial(
        pl.kernel,
        mesh=mesh,
        compiler_params=pltpu.CompilerParams(needs_layout_passes=False),
        out_type=jax.ShapeDtypeStruct((M, T, _DP), jnp.float32),
        scratch_types=[
            pltpu.VMEM((T,), jnp.int32),
            pltpu.VMEM((T,), jnp.int32),
            pltpu.VMEM((T,), jnp.int32),
            pltpu.VMEM((K,), jnp.int32),
            pltpu.VMEM((T,), jnp.int32),
            pltpu.VMEM((_CH, _DP), jnp.float32),
            pltpu.VMEM((_CH, _DP), jnp.float32),
            pltpu.SemaphoreType.DMA,
            pltpu.SemaphoreType.DMA,
            pltpu.SemaphoreType.DMA,
            pltpu.SemaphoreType.DMA,
        ],
    )
    def sc_kernel(qk_hbm, kk_hbm, vk_hbm, ef_hbm, out_hbm,
                  qv, kv, vv, tbl, idxv, rows0, rows1,
                  gsem0, gsem1, wsem0, wsem1):
        w = lax.axis_index("s") * _NC + lax.axis_index("c")

        @pl.when(w < M)
        def _():
            m = w
            pltpu.sync_copy(qk_hbm.at[m], qv)
            pltpu.sync_copy(kk_hbm.at[m], kv)
            pltpu.sync_copy(vk_hbm.at[m], vv)

            neg = jnp.full((16,), -1, jnp.int32)

            def init_step(i, carry):
                tbl[pl.ds(i * 16, 16)] = neg
                return carry

            lax.fori_loop(0, K // 16, init_step, 0)

            base = m * RP
            iota = lax.iota(jnp.int32, 16)

            def scan_step(ci, carry):
                t0 = ci * 16
                qv16 = qv[pl.ds(t0, 16)]
                kv16 = kv[pl.ds(t0, 16)]
                tau_prev = plsc.load_gather(tbl, [qv16])
                qc = []
                kc = []
                for j in range(16):
                    kjb = jnp.full((16,), kv16[j])
                    qc.append(jnp.where((qv16 == kjb) & (iota >= j), t0 + j, -1))
                    kc.append(jnp.where(kv16 == kjb, j, -1))
                while len(qc) > 1:
                    qc = [jnp.maximum(a, b) for a, b in zip(qc[::2], qc[1::2])]
                    kc = [jnp.maximum(a, b) for a, b in zip(kc[::2], kc[1::2])]
                tau = jnp.maximum(qc[0], tau_prev)
                plsc.store_scatter(tbl, [kv16], t0 + iota, mask=kc[0] == iota)
                valid = tau >= 0
                vi = plsc.load_gather(vv, [jnp.maximum(tau, 0)])
                idxv[pl.ds(t0, 16)] = jnp.where(valid, base + vi + 1, base)
                return carry

            lax.fori_loop(0, T // 16, scan_step, 0, unroll=2)

            nch = T // _CH
            bufs = (rows0, rows1)
            gsems = (gsem0, gsem1)
            wsems = (wsem0, wsem1)

            def fire_gather(ci):
                return pltpu.async_copy(
                    ef_hbm.at[idxv.at[pl.ds(ci * _CH, _CH)]],
                    bufs[ci % 2], gsems[ci % 2])

            def fire_write(ci):
                return pltpu.async_copy(
                    bufs[ci % 2], out_hbm.at[m, pl.ds(ci * _CH, _CH)],
                    wsems[ci % 2])

            gcop = [None] * nch
            wcop = [None] * nch
            gcop[0] = fire_gather(0)
            for ci in range(nch):
                gcop[ci].wait()
                wcop[ci] = fire_write(ci)
                if ci + 1 < nch:
                    if ci >= 1:
                        wcop[ci - 1].wait()
                    gcop[ci + 1] = fire_gather(ci + 1)
            wcop[nch - 1].wait()

    return sc_kernel(qk, kk, vk, ef)


def kernel(x_ln1, v_first, Wq, Wk, Wv, E_v, W_att):
    x = x_ln1[0]
    T, C = x.shape
    M, D, K = Wq.shape
    KV = Wv.shape[2]
    qk, kk, vk = _compute_codes(x, Wq, Wk, Wv)
    RP = (KV + 1 + 7) // 8 * 8
    ef = _prep_ef(E_v, RP)
    vout = _sc_vout(qk, kk, vk, ef, T, C, M, D, K, RP)
    y = _compute_out(x, W_att, vout)
    return (y[None], v_first)


# speedup vs baseline: 40.2387x; 1.5038x over previous
import functools

import jax
import jax.numpy as jnp
from jax import lax
from jax.experimental import pallas as pl
from jax.experimental.pallas import tpu as pltpu
from jax.experimental.pallas import tpu_sc as plsc

_TT = 256
_CH = 128
_NC = 2
_NS = 16
_DP = 128


def _codes_body(x_ref, wq_ref, wk_ref, wv_ref, q_ref, k_ref, v_ref):
    TT, C = x_ref.shape
    M, D, K = wq_ref.shape
    CK = 128
    rows_f = lax.broadcasted_iota(jnp.int32, (CK, TT), 0).astype(jnp.float32)

    def fold(v, ix, n, stop):
        while n > stop:
            h = n // 2
            keep = v[:h] >= v[h:n]
            v = jnp.where(keep, v[:h], v[h:n])
            ix = jnp.where(keep, ix[:h], ix[h:n])
            n = h
        return v, ix

    for m in range(M):
        u = x_ref[:, m * D:(m + 1) * D]
        for w_ref, o_ref in ((wq_ref, q_ref), (wk_ref, k_ref), (wv_ref, v_ref)):
            pv, pi = None, None
            for c in range(K // CK):
                logits = lax.dot_general(w_ref[m, :, c * CK:(c + 1) * CK], u,
                                         (((0,), (1,)), ((), ())),
                                         preferred_element_type=jnp.float32)
                v, ix = fold(logits, rows_f + float(c * CK), CK, 8)
                if pv is None:
                    pv, pi = v, ix
                else:
                    keep = pv >= v
                    pv = jnp.where(keep, pv, v)
                    pi = jnp.where(keep, pi, ix)
            v, ix = fold(pv, pi, 8, 1)
            o_ref[m, :] = ix[0].astype(jnp.int32)


def _compute_codes(x, Wq, Wk, Wv):
    T, C = x.shape
    M, D, K = Wq.shape
    grid = (T // _TT,)
    wspec = pl.BlockSpec((M, D, K), lambda i: (0, 0, 0))
    ospec = pl.BlockSpec((16, _TT), lambda i: (0, i))
    oshape = jax.ShapeDtypeStruct((16, T), jnp.int32)
    return pl.pallas_call(
        _codes_body,
        grid=grid,
        in_specs=[pl.BlockSpec((_TT, C), lambda i: (i, 0)), wspec, wspec, wspec],
        out_specs=[ospec, ospec, ospec],
        out_shape=[oshape, oshape, oshape],
    )(x, Wq, Wk, Wv)


def _out_body(x_ref, w_ref, v_ref, o_ref):
    M, TT, DP = v_ref.shape
    C = x_ref.shape[1]
    D = C // M
    y = lax.dot_general(x_ref[...], w_ref[...], (((1,), (0,)), ((), ())),
                        preferred_element_type=jnp.float32)
    for m in range(M):
        o_ref[:, m * D:(m + 1) * D] = y[:, m * D:(m + 1) * D] + v_ref[m][:, :D]


def _compute_out(x, W_att, vout):
    T, C = x.shape
    M, _, DP = vout.shape
    return pl.pallas_call(
        _out_body,
        grid=(T // _TT,),
        in_specs=[pl.BlockSpec((_TT, C), lambda i: (i, 0)),
                  pl.BlockSpec((C, C), lambda i: (0, 0)),
                  pl.BlockSpec((M, _TT, DP), lambda i: (0, i, 0))],
        out_specs=pl.BlockSpec((_TT, C), lambda i: (i, 0)),
        out_shape=jax.ShapeDtypeStruct((T, C), jnp.float32),
    )(x, W_att, vout)


def _sc_vout(qk, kk, vk, E_v, T, C, M, D, K, KV):
    mesh = plsc.VectorSubcoreMesh(core_axis_name="c", subcore_axis_name="s")

    @functools.partial(
        pl.kernel,
        mesh=mesh,
        compiler_params=pltpu.CompilerParams(needs_layout_passes=False),
        out_type=jax.ShapeDtypeStruct((M, T, D), jnp.float32),
        scratch_types=[
            pltpu.VMEM((T,), jnp.int32),
            pltpu.VMEM((T,), jnp.int32),
            pltpu.VMEM((T,), jnp.int32),
            pltpu.VMEM((K,), jnp.int32),
            pltpu.VMEM((T,), jnp.int32),
            pltpu.VMEM(((KV + 1) * D,), jnp.float32),
            pltpu.VMEM((_CH, 64), jnp.float32),
            pltpu.VMEM((_CH, 64), jnp.float32),
            pltpu.SemaphoreType.DMA,
            pltpu.SemaphoreType.DMA,
        ],
    )
    def sc_kernel(qk_hbm, kk_hbm, vk_hbm, ev_hbm, out_hbm,
                  qv, kv, vv, tbl, idxv, emb, obuf0, obuf1,
                  wsem0, wsem1):
        w = lax.axis_index("s") * _NC + lax.axis_index("c")

        @pl.when(w < M)
        def _():
            m = w
            nw = (KV + 1) * D
            pltpu.sync_copy(qk_hbm.at[m], qv)
            pltpu.sync_copy(kk_hbm.at[m], kv)
            pltpu.sync_copy(vk_hbm.at[m], vv)
            pltpu.sync_copy(ev_hbm.at[pl.ds(m * nw, nw)], emb)

            neg = jnp.full((16,), -1, jnp.int32)

            def init_step(i, carry):
                tbl[pl.ds(i * 16, 16)] = neg
                return carry

            lax.fori_loop(0, K // 16, init_step, 0)

            iota = lax.iota(jnp.int32, 16)

            def scan_step(ci, carry):
                t0 = ci * 16
                qv16 = qv[pl.ds(t0, 16)]
                kv16 = kv[pl.ds(t0, 16)]
                tau_prev = plsc.load_gather(tbl, [qv16])
                qc = []
                kc = []
                for j in range(16):
                    kjb = jnp.full((16,), kv16[j])
                    qc.append(jnp.where((qv16 == kjb) & (iota >= j), t0 + j, -1))
                    kc.append(jnp.where(kv16 == kjb, j, -1))
                while len(qc) > 1:
                    qc = [jnp.maximum(a, b) for a, b in zip(qc[::2], qc[1::2])]
                    kc = [jnp.maximum(a, b) for a, b in zip(kc[::2], kc[1::2])]
                tau = jnp.maximum(qc[0], tau_prev)
                plsc.store_scatter(tbl, [kv16], t0 + iota, mask=kc[0] == iota)
                valid = tau >= 0
                vi = plsc.load_gather(vv, [jnp.maximum(tau, 0)])
                idxv[pl.ds(t0, 16)] = jnp.where(valid, vi + 1, 0)
                return carry

            lax.fori_loop(0, T // 16, scan_step, 0, unroll=2)

            nch = T // _CH
            bufs = (obuf0, obuf1)
            wsems = (wsem0, wsem1)
            cols = [iota + g * 16 for g in range(4)]
            wcop = [None] * nch
            for ci in range(nch):
                ob = bufs[ci % 2]
                if ci >= 2:
                    wcop[ci - 2].wait()

                def group_step(gi, carry, ob=ob, ci=ci):
                    t0 = ci * _CH + gi * 16
                    idx16 = idxv[pl.ds(t0, 16)] * D
                    for l in range(16):
                        rowb = jnp.full((16,), idx16[l])
                        for g in range(4):
                            val = plsc.load_gather(emb, [rowb + cols[g]])
                            ob[gi * 16 + l, pl.ds(g * 16, 16)] = val
                    return carry

                lax.fori_loop(0, _CH // 16, group_step, 0)
                wcop[ci] = pltpu.async_copy(
                    ob, out_hbm.at[m, pl.ds(ci * _CH, _CH)], wsems[ci % 2])
            wcop[nch - 2].wait()
            wcop[nch - 1].wait()

    return sc_kernel(qk, kk, vk, E_v.reshape(M * (KV + 1) * D))


def kernel(x_ln1, v_first, Wq, Wk, Wv, E_v, W_att):
    B, T, C = x_ln1.shape
    x = x_ln1.reshape(T, C)
    M, D, K = Wq.shape
    KV = Wv.shape[2]
    qk, kk, vk = _compute_codes(x, Wq, Wk, Wv)
    vout = _sc_vout(qk, kk, vk, E_v, T, C, M, D, K, KV)
    y = _compute_out(x, W_att, vout)
    return (y.reshape(B, T, C), v_first)
